# baseline (speedup 1.0000x reference)
import functools

import jax
import jax.numpy as jnp
from jax import lax
from jax.experimental import pallas as pl
from jax.experimental.pallas import tpu as pltpu
from jax.experimental.pallas import tpu_sc as plsc

VOCAB = 250002
DIM = 1024
B = 2
S = 4096

NC = 2
NS = 16
NW = NC * NS
N = B * S
PER_W = N // NW
W_PER_ROW = S // PER_W
LANES = 16
NCOL = DIM // LANES

SIZES = [64, 56, 64, 56, 16]
OFFS = [0, 64, 120, 184, 240]
NCHUNK = len(SIZES)
SROWS = 16

_mesh = plsc.VectorSubcoreMesh(core_axis_name="c", subcore_axis_name="s")


@functools.partial(
    pl.kernel,
    mesh=_mesh,
    out_type=jax.ShapeDtypeStruct((B, S, DIM), jnp.float32),
    scratch_types=[
        pltpu.VMEM((PER_W,), jnp.int32),
        pltpu.VMEM((DIM,), jnp.float32),
        pltpu.VMEM((64, DIM), jnp.float32),
        pltpu.VMEM((56, DIM), jnp.float32),
        pltpu.SemaphoreType.DMA((2,)),
        pltpu.SemaphoreType.DMA((2,)),
    ],
)
def _embed(ids_hbm, tt_hbm, table_hbm, out_hbm, idx_v, tt_v, buf_a, buf_b,
           gsem, osem):
    wid = lax.axis_index("s") * NC + lax.axis_index("c")
    brow = wid // W_PER_ROW
    col0 = (wid % W_PER_ROW) * PER_W
    bufs = [buf_a, buf_b]
    pltpu.sync_copy(ids_hbm.at[brow, pl.ds(col0, PER_W)], idx_v)

    def gather(c):
        b = c % 2
        return pltpu.async_copy(
            table_hbm.at[idx_v.at[pl.ds(OFFS[c], SIZES[c])]],
            bufs[b].at[pl.ds(0, SIZES[c])],
            gsem.at[b],
        )

    def scatter_wait(c):
        b = c % 2
        pltpu.make_async_copy(
            bufs[b].at[pl.ds(0, SIZES[c])],
            out_hbm.at[brow, pl.ds(col0, SIZES[c])],
            osem.at[b],
        ).wait()

    def add_and_scatter(c):
        b = c % 2
        buf = bufs[b]
        for r0 in range(0, SIZES[c], SROWS):
            nr = min(SROWS, SIZES[c] - r0)

            @plsc.parallel_loop(0, NCOL)
            def _(j):
                ttv = tt_v[pl.ds(j * LANES, LANES)]
                for i in range(r0, r0 + nr):
                    plsc.addupdate(buf.at[i, pl.ds(j * LANES, LANES)], ttv)

            pltpu.async_copy(
                buf.at[pl.ds(r0, nr)],
                out_hbm.at[brow, pl.ds(col0 + OFFS[c] + r0, nr)],
                osem.at[b],
            )

    gathers = [None] * NCHUNK
    gathers[0] = gather(0)
    pltpu.sync_copy(tt_hbm.at[0], tt_v)
    for c in range(NCHUNK):
        gathers[c].wait()
        if c + 1 < NCHUNK:
            if c >= 1:
                scatter_wait(c - 1)
            gathers[c + 1] = gather(c + 1)
        add_and_scatter(c)
    scatter_wait(NCHUNK - 2)
    scatter_wait(NCHUNK - 1)


def kernel(input_ids, word_table, token_type_table):
    return _embed(input_ids.astype(jnp.int32), token_type_table, word_table)
